# Initial kernel scaffold; baseline (speedup 1.0000x reference)
import jax
import jax.numpy as jnp
from jax.experimental import pallas as pl


def kernel(positions, hidden_states, w_in_ln, w_post_ln, wq, wk, wv, wo, wg, expert_bias, w_gate_up, w_down, sh_gate_up, sh_down):
    raise NotImplementedError("write your pallas kernel here")


# speedup vs baseline: 1.2936x; 1.2936x over previous
import functools
import jax
import jax.numpy as jnp
from jax.experimental import pallas as pl
from jax.experimental.pallas import tpu as pltpu

T = 2048; D = 1024; H = 16; KVH = 4; HD = 64; E = 8; TK = 2; FF = 512
EPS = 1e-6; THETA = 10000.0
BT = 256
NT = T // BT
HALF = HD // 2


def _rms(x, w):
    return x * jax.lax.rsqrt(jnp.mean(x * x, axis=-1, keepdims=True) + EPS) * w


def _qkv_body(h_ref, ln_ref, wq_ref, wk_ref, wv_ref, cos_ref, sin_ref,
              q_ref, k_ref, v_ref):
    x = _rms(h_ref[...], ln_ref[0:1, :])
    q = jnp.dot(x, wq_ref[...], preferred_element_type=jnp.float32)
    k = jnp.dot(x, wk_ref[...], preferred_element_type=jnp.float32)
    v = jnp.dot(x, wv_ref[...], preferred_element_type=jnp.float32)
    cos = cos_ref[...]
    sin = sin_ref[...]

    parts = []
    for h in range(H):
        x1 = q[:, h * HD:h * HD + HALF]
        x2 = q[:, h * HD + HALF:(h + 1) * HD]
        parts.append(x1 * cos - x2 * sin)
        parts.append(x2 * cos + x1 * sin)
    q_ref[...] = jnp.concatenate(parts, axis=1)
    for h in range(KVH):
        x1 = k[:, h * HD:h * HD + HALF]
        x2 = k[:, h * HD + HALF:(h + 1) * HD]
        k_ref[h] = jnp.concatenate([x1 * cos - x2 * sin, x2 * cos + x1 * sin],
                                   axis=1)
        v_ref[h] = v[:, h * HD:(h + 1) * HD]


def _qkv_call(hidden, w_in_ln, wq, wk, wv, cos, sin):
    ln2 = jnp.broadcast_to(w_in_ln[None, :], (8, D))
    return pl.pallas_call(
        _qkv_body,
        grid=(NT,),
        in_specs=[
            pl.BlockSpec((BT, D), lambda i: (i, 0)),
            pl.BlockSpec((8, D), lambda i: (0, 0)),
            pl.BlockSpec((D, H * HD), lambda i: (0, 0)),
            pl.BlockSpec((D, KVH * HD), lambda i: (0, 0)),
            pl.BlockSpec((D, KVH * HD), lambda i: (0, 0)),
            pl.BlockSpec((BT, HALF), lambda i: (i, 0)),
            pl.BlockSpec((BT, HALF), lambda i: (i, 0)),
        ],
        out_specs=[
            pl.BlockSpec((BT, H * HD), lambda i: (i, 0)),
            pl.BlockSpec((KVH, BT, HD), lambda i: (0, i, 0)),
            pl.BlockSpec((KVH, BT, HD), lambda i: (0, i, 0)),
        ],
        out_shape=[
            jax.ShapeDtypeStruct((T, H * HD), jnp.float32),
            jax.ShapeDtypeStruct((KVH, T, HD), jnp.float32),
            jax.ShapeDtypeStruct((KVH, T, HD), jnp.float32),
        ],
    )(hidden, ln2, wq, wk, wv, cos, sin)


def _attn_body(q_ref, k_ref, v_ref, o_ref):
    qi = pl.program_id(1)
    k = k_ref[0]
    v = v_ref[0]
    row = qi * BT + jax.lax.broadcasted_iota(jnp.int32, (BT, T), 0)
    col = jax.lax.broadcasted_iota(jnp.int32, (BT, T), 1)
    causal = col <= row
    outs = []
    for a in range(2):
        q = q_ref[:, a * HD:(a + 1) * HD]
        s = jax.lax.dot_general(q, k, (((1,), (1,)), ((), ())),
                                preferred_element_type=jnp.float32)
        s = s * (HD ** -0.5)
        s = jnp.where(causal, s, -1e30)
        m = jnp.max(s, axis=1, keepdims=True)
        p = jnp.exp(s - m)
        l = jnp.sum(p, axis=1, keepdims=True)
        o = jnp.dot(p, v, preferred_element_type=jnp.float32)
        outs.append(o / l)
    o_ref[...] = jnp.concatenate(outs, axis=1)


def _attn_call(q, k, v):
    return pl.pallas_call(
        _attn_body,
        grid=(H // 2, NT),
        in_specs=[
            pl.BlockSpec((BT, 2 * HD), lambda j, i: (i, j)),
            pl.BlockSpec((1, T, HD), lambda j, i: (j // 2, 0, 0)),
            pl.BlockSpec((1, T, HD), lambda j, i: (j // 2, 0, 0)),
        ],
        out_specs=pl.BlockSpec((BT, 2 * HD), lambda j, i: (i, j)),
        out_shape=jax.ShapeDtypeStruct((T, H * HD), jnp.float32),
    )(q, k, v)


def _oproj_body(a_ref, h_ref, wo_ref, ln_ref, wg_ref, bias_ref,
                h2_ref, xn_ref, topi_ref, wts_ref):
    att = jnp.dot(a_ref[...], wo_ref[...], preferred_element_type=jnp.float32)
    h2 = h_ref[...] + att
    h2_ref[...] = h2
    xn = _rms(h2, ln_ref[0:1, :])
    xn_ref[...] = xn
    logits = jnp.dot(xn, wg_ref[...], preferred_element_type=jnp.float32)
    s = jax.nn.sigmoid(logits)
    c = s + bias_ref[0:1, :]
    iota = jax.lax.broadcasted_iota(jnp.int32, (BT, E), 1)
    m1 = jnp.max(c, axis=1, keepdims=True)
    i1 = jnp.min(jnp.where(c == m1, iota, E), axis=1, keepdims=True)
    c2 = jnp.where(iota == i1, -jnp.inf, c)
    m2 = jnp.max(c2, axis=1, keepdims=True)
    i2 = jnp.min(jnp.where(c2 == m2, iota, E), axis=1, keepdims=True)
    w1 = jnp.sum(jnp.where(iota == i1, s, 0.0), axis=1, keepdims=True)
    w2 = jnp.sum(jnp.where(iota == i2, s, 0.0), axis=1, keepdims=True)
    tot = w1 + w2
    topi_ref[...] = jnp.concatenate([i1, i2], axis=1)
    wts_ref[...] = jnp.concatenate([w1 / tot, w2 / tot], axis=1)


def _oproj_call(attn, hidden, wo, w_post_ln, wg, expert_bias):
    ln2 = jnp.broadcast_to(w_post_ln[None, :], (8, D))
    bias2 = jnp.broadcast_to(expert_bias[None, :], (8, E))
    return pl.pallas_call(
        _oproj_body,
        grid=(NT,),
        in_specs=[
            pl.BlockSpec((BT, H * HD), lambda i: (i, 0)),
            pl.BlockSpec((BT, D), lambda i: (i, 0)),
            pl.BlockSpec((H * HD, D), lambda i: (0, 0)),
            pl.BlockSpec((8, D), lambda i: (0, 0)),
            pl.BlockSpec((D, E), lambda i: (0, 0)),
            pl.BlockSpec((8, E), lambda i: (0, 0)),
        ],
        out_specs=[
            pl.BlockSpec((BT, D), lambda i: (i, 0)),
            pl.BlockSpec((BT, D), lambda i: (i, 0)),
            pl.BlockSpec((BT, TK), lambda i: (i, 0)),
            pl.BlockSpec((BT, TK), lambda i: (i, 0)),
        ],
        out_shape=[
            jax.ShapeDtypeStruct((T, D), jnp.float32),
            jax.ShapeDtypeStruct((T, D), jnp.float32),
            jax.ShapeDtypeStruct((T, TK), jnp.int32),
            jax.ShapeDtypeStruct((T, TK), jnp.float32),
        ],
    )(attn, hidden, wo, ln2, wg, bias2)


def _moe_body(xn_ref, wgu_ref, wdn_ref, topi_ref, wts_ref, res_ref, out_ref):
    e = pl.program_id(1)
    xn = xn_ref[...]
    gu = jnp.dot(xn, wgu_ref[0], preferred_element_type=jnp.float32)
    g = gu[:, :FF]
    u = gu[:, FF:]
    o = jnp.dot(g * jax.nn.sigmoid(g) * u, wdn_ref[0],
                preferred_element_type=jnp.float32)
    topi = topi_ref[...]
    wts = wts_ref[...]
    we = jnp.sum(jnp.where(topi == e, wts, 0.0), axis=1, keepdims=True)
    we = jnp.where(e == E, 1.0, we)
    contrib = we * o

    @pl.when(e == 0)
    def _():
        out_ref[...] = res_ref[...] + contrib

    @pl.when(e != 0)
    def _():
        out_ref[...] += contrib


def _moe_call(xn, wgu9, wdn9, topi, wts, res2):
    return pl.pallas_call(
        _moe_body,
        grid=(NT, E + 1),
        in_specs=[
            pl.BlockSpec((BT, D), lambda i, e: (i, 0)),
            pl.BlockSpec((1, D, 2 * FF), lambda i, e: (e, 0, 0)),
            pl.BlockSpec((1, FF, D), lambda i, e: (e, 0, 0)),
            pl.BlockSpec((BT, TK), lambda i, e: (i, 0)),
            pl.BlockSpec((BT, TK), lambda i, e: (i, 0)),
            pl.BlockSpec((BT, D), lambda i, e: (i, 0)),
        ],
        out_specs=pl.BlockSpec((BT, D), lambda i, e: (i, 0)),
        out_shape=jax.ShapeDtypeStruct((T, D), jnp.float32),
    )(xn, wgu9, wdn9, topi, wts, res2)


def kernel(positions, hidden_states, w_in_ln, w_post_ln, wq, wk, wv, wo, wg,
           expert_bias, w_gate_up, w_down, sh_gate_up, sh_down):
    inv = 1.0 / (THETA ** (jnp.arange(0, HALF, dtype=jnp.float32) * 2.0 / HD))
    ang = positions.astype(jnp.float32)[:, None] * inv[None, :]
    cos = jnp.cos(ang)
    sin = jnp.sin(ang)

    q, k, v = _qkv_call(hidden_states, w_in_ln, wq, wk, wv, cos, sin)
    attn = _attn_call(q, k, v)
    h2, xn, topi, wts = _oproj_call(attn, hidden_states, wo, w_post_ln, wg,
                                    expert_bias)
    wgu9 = jnp.concatenate([w_gate_up, sh_gate_up[None]], axis=0)
    wdn9 = jnp.concatenate([w_down, sh_down[None]], axis=0)
    out = _moe_call(xn, wgu9, wdn9, topi, wts, h2)
    return out
